# Initial kernel scaffold; baseline (speedup 1.0000x reference)
import functools

import jax
import jax.numpy as jnp
from jax import lax
from jax.experimental import pallas as pl
from jax.experimental.pallas import tpu as pltpu
from jax.experimental.pallas import tpu_sc as plsc

N = 10000
NP = 10240
E = 160000
EP = 163840
D = 256
H = 512
NCORE = 2
NSUB = 16
ROWS_PER_TILE = NP // NSUB
CHUNK = 128

_mesh = plsc.VectorSubcoreMesh(core_axis_name="c", subcore_axis_name="s")


def _deg_body(dst_hbm, zeros16, ones16, out_hbm, dst_v, ones_v, acc_sh, sem):
    c = lax.axis_index("c")
    s = lax.axis_index("s")
    r0 = s * ROWS_PER_TILE
    pltpu.sync_copy(dst_hbm.at[c].at[s], dst_v)
    pltpu.sync_copy(ones16, ones_v)
    pltpu.sync_copy(zeros16.at[pl.ds(r0, ROWS_PER_TILE)],
                    acc_sh.at[pl.ds(r0, ROWS_PER_TILE)])
    plsc.subcore_barrier()

    def chunk(j, carry):
        pltpu.sync_copy(ones_v, acc_sh.at[dst_v.at[j]], add=True)
        return carry

    lax.fori_loop(0, 40, chunk, 0)
    plsc.subcore_barrier()
    pltpu.sync_copy(acc_sh.at[pl.ds(r0, ROWS_PER_TILE)],
                    out_hbm.at[c].at[pl.ds(r0, ROWS_PER_TILE)])


_deg_kernel = functools.partial(
    pl.kernel,
    out_type=jax.ShapeDtypeStruct((NCORE, NP, 16), jnp.float32),
    mesh=_mesh,
    scratch_types=[
        pltpu.VMEM((40, CHUNK), jnp.int32),
        pltpu.VMEM((CHUNK, 16), jnp.float32),
        pltpu.VMEM_SHARED((NP, 16), jnp.float32),
        pltpu.SemaphoreType.DMA,
    ],
)(_deg_body)


def _make_segsum(groups):
    gpc = groups // NCORE

    def body(src_hbm, dst_hbm, zeros128, table_hbm, out_hbm,
             src_v, dst_v, buf, acc_sh, sem):
        c = lax.axis_index("c")
        s = lax.axis_index("s")
        r0 = s * ROWS_PER_TILE
        pltpu.sync_copy(src_hbm.at[s], src_v)
        pltpu.sync_copy(dst_hbm.at[s], dst_v)
        for i in range(gpc):
            g = c * gpc + i
            pltpu.sync_copy(zeros128.at[pl.ds(r0, ROWS_PER_TILE)],
                            acc_sh.at[pl.ds(r0, ROWS_PER_TILE)])
            plsc.subcore_barrier()

            def chunk(j, carry):
                pltpu.async_copy(table_hbm.at[g].at[src_v.at[j]], buf,
                                 sem).wait()
                pltpu.sync_copy(buf, acc_sh.at[dst_v.at[j]], add=True)
                return carry

            lax.fori_loop(0, 80, chunk, 0)
            plsc.subcore_barrier()
            pltpu.sync_copy(acc_sh.at[pl.ds(r0, ROWS_PER_TILE)],
                            out_hbm.at[g].at[pl.ds(r0, ROWS_PER_TILE)])

    return functools.partial(
        pl.kernel,
        out_type=jax.ShapeDtypeStruct((groups, NP, 128), jnp.float32),
        mesh=_mesh,
        scratch_types=[
            pltpu.VMEM((80, CHUNK), jnp.int32),
            pltpu.VMEM((80, CHUNK), jnp.int32),
            pltpu.VMEM((CHUNK, 128), jnp.float32),
            pltpu.VMEM_SHARED((NP, 128), jnp.float32),
            pltpu.SemaphoreType.DMA,
        ],
    )(body)


_segsum2 = _make_segsum(2)
_segsum4 = _make_segsum(4)


def _head_gather_body(src_hbm, dst_hbm, a_hbm, b_hbm, z_hbm,
                      src_v, dst_v, buf, sem_a, sem_b):
    c = lax.axis_index("c")
    s = lax.axis_index("s")
    base = (c * NSUB + s) * 40 * CHUNK
    pltpu.sync_copy(src_hbm.at[c].at[s], src_v)
    pltpu.sync_copy(dst_hbm.at[c].at[s], dst_v)

    def chunk(j, carry):
        pltpu.async_copy(a_hbm.at[src_v.at[j]], buf, sem_a).wait()
        pltpu.async_copy(b_hbm.at[dst_v.at[j]], buf, sem_b, add=True).wait()
        pltpu.sync_copy(buf, z_hbm.at[pl.ds(base + j * CHUNK, CHUNK)])
        return carry

    lax.fori_loop(0, 40, chunk, 0)


_head_gather = functools.partial(
    pl.kernel,
    out_type=jax.ShapeDtypeStruct((EP, H), jnp.float32),
    mesh=_mesh,
    scratch_types=[
        pltpu.VMEM((40, CHUNK), jnp.int32),
        pltpu.VMEM((40, CHUNK), jnp.int32),
        pltpu.VMEM((CHUNK, H), jnp.float32),
        pltpu.SemaphoreType.DMA,
        pltpu.SemaphoreType.DMA,
    ],
)(_head_gather_body)


def _dinv_from_parts(dp):
    deg = dp[0, :, 0:1] + dp[1, :, 0:1] + 1.0
    return lax.rsqrt(jnp.maximum(deg, 1.0))


def _xs1_body(dp_ref, x_ref, out_ref):
    dinv = _dinv_from_parts(dp_ref[...])
    xs = x_ref[...] * dinv
    out_ref[0] = xs[:, 0:128]
    out_ref[1] = xs[:, 128:256]


def _mm1_body(dp_ref, s1_ref, x1_ref, w_ref, b_ref, out_ref):
    dinv = _dinv_from_parts(dp_ref[...])
    s1 = s1_ref[...]
    x1 = x1_ref[...]
    t = jnp.concatenate([dinv * (s1[0] + x1[0]), dinv * (s1[1] + x1[1])],
                        axis=1)
    h = jnp.maximum(jnp.dot(t, w_ref[...],
                            preferred_element_type=jnp.float32) + b_ref[...],
                    0.0)
    x2 = h * dinv
    for g in range(4):
        out_ref[g] = x2[:, 128 * g:128 * (g + 1)]


def _mm2_body(dp_ref, s2_ref, x2_ref, w_ref, b_ref, p1_ref, pb1_ref,
              a_ref, bb_ref):
    dinv = _dinv_from_parts(dp_ref[...])
    s2 = s2_ref[...]
    x2 = x2_ref[...]
    t = jnp.concatenate([dinv * (s2[g] + x2[g]) for g in range(4)], axis=1)
    emb = jnp.dot(t, w_ref[...], preferred_element_type=jnp.float32) + b_ref[...]
    p1 = p1_ref[...]
    a_ref[...] = jnp.dot(emb, p1[0:H], preferred_element_type=jnp.float32) \
        + pb1_ref[...]
    bb_ref[...] = jnp.dot(emb, p1[H:2 * H], preferred_element_type=jnp.float32)


def _headmv_body(z_ref, p2_ref, pb2_ref, out_ref):
    z = jnp.maximum(z_ref[...], 0.0)
    out_ref[...] = jnp.dot(z, p2_ref[...],
                           preferred_element_type=jnp.float32) + pb2_ref[...]


_RB = 256
_GRID_N = NP // _RB


def _tc_xs1(deg_parts, x_pad):
    return pl.pallas_call(
        _xs1_body,
        grid=(_GRID_N,),
        in_specs=[
            pl.BlockSpec((NCORE, _RB, 16), lambda i: (0, i, 0)),
            pl.BlockSpec((_RB, D), lambda i: (i, 0)),
        ],
        out_specs=pl.BlockSpec((2, _RB, 128), lambda i: (0, i, 0)),
        out_shape=jax.ShapeDtypeStruct((2, NP, 128), jnp.float32),
    )(deg_parts, x_pad)


def _tc_mm1(deg_parts, s1, x1, W1, b1):
    return pl.pallas_call(
        _mm1_body,
        grid=(_GRID_N,),
        in_specs=[
            pl.BlockSpec((NCORE, _RB, 16), lambda i: (0, i, 0)),
            pl.BlockSpec((2, _RB, 128), lambda i: (0, i, 0)),
            pl.BlockSpec((2, _RB, 128), lambda i: (0, i, 0)),
            pl.BlockSpec((D, H), lambda i: (0, 0)),
            pl.BlockSpec((1, H), lambda i: (0, 0)),
        ],
        out_specs=pl.BlockSpec((4, _RB, 128), lambda i: (0, i, 0)),
        out_shape=jax.ShapeDtypeStruct((4, NP, 128), jnp.float32),
    )(deg_parts, s1, x1, W1, b1)


def _tc_mm2(deg_parts, s2, x2, W2, b2, P1, pb1):
    return pl.pallas_call(
        _mm2_body,
        grid=(_GRID_N,),
        in_specs=[
            pl.BlockSpec((NCORE, _RB, 16), lambda i: (0, i, 0)),
            pl.BlockSpec((4, _RB, 128), lambda i: (0, i, 0)),
            pl.BlockSpec((4, _RB, 128), lambda i: (0, i, 0)),
            pl.BlockSpec((H, H), lambda i: (0, 0)),
            pl.BlockSpec((1, H), lambda i: (0, 0)),
            pl.BlockSpec((2 * H, H), lambda i: (0, 0)),
            pl.BlockSpec((1, H), lambda i: (0, 0)),
        ],
        out_specs=[
            pl.BlockSpec((_RB, H), lambda i: (i, 0)),
            pl.BlockSpec((_RB, H), lambda i: (i, 0)),
        ],
        out_shape=[
            jax.ShapeDtypeStruct((NP, H), jnp.float32),
            jax.ShapeDtypeStruct((NP, H), jnp.float32),
        ],
    )(deg_parts, s2, x2, W2, b2, P1, pb1)


_EB = 1024


def _tc_head(Z, P2, pb2):
    return pl.pallas_call(
        _headmv_body,
        grid=(EP // _EB,),
        in_specs=[
            pl.BlockSpec((_EB, H), lambda i: (i, 0)),
            pl.BlockSpec((H, 1), lambda i: (0, 0)),
            pl.BlockSpec((1, 1), lambda i: (0, 0)),
        ],
        out_specs=pl.BlockSpec((_EB, 1), lambda i: (i, 0)),
        out_shape=jax.ShapeDtypeStruct((EP, 1), jnp.float32),
    )(Z, P2, pb2)


def kernel(x, edge_index, W1, b1, W2, b2, P1, pb1, P2, pb2):
    ei = edge_index.astype(jnp.int32)
    src = jnp.concatenate([ei[0], jnp.zeros((EP - E,), jnp.int32)])
    dst = jnp.concatenate([ei[1], jnp.full((EP - E,), N, jnp.int32)])

    src32 = src.reshape(NCORE, NSUB, 40, CHUNK)
    dst32 = dst.reshape(NCORE, NSUB, 40, CHUNK)
    src16 = src.reshape(NSUB, 80, CHUNK)
    dst16 = dst.reshape(NSUB, 80, CHUNK)

    x_pad = jnp.zeros((NP, D), x.dtype).at[:N].set(x)
    zeros16 = jnp.zeros((NP, 16), jnp.float32)
    zeros128 = jnp.zeros((NP, 128), jnp.float32)
    ones16 = jnp.ones((CHUNK, 16), jnp.float32)

    deg_parts = _deg_kernel(dst32, zeros16, ones16)
    xs1 = _tc_xs1(deg_parts, x_pad)
    s1 = _segsum2(src16, dst16, zeros128, xs1)
    xs2 = _tc_mm1(deg_parts, s1, xs1, W1, b1.reshape(1, H))
    s2 = _segsum4(src16, dst16, zeros128, xs2)
    A, B = _tc_mm2(deg_parts, s2, xs2, W2, b2.reshape(1, H), P1,
                   pb1.reshape(1, H))
    Z = _head_gather(src32, dst32, A, B)
    pred = _tc_head(Z, P2, pb2.reshape(1, 1))
    return pred[:E, 0]


# speedup vs baseline: 1.6900x; 1.6900x over previous
import functools

import jax
import jax.numpy as jnp
from jax import lax
from jax.experimental import pallas as pl
from jax.experimental.pallas import tpu as pltpu
from jax.experimental.pallas import tpu_sc as plsc

N = 10000
NP = 10240
E = 160000
EP = 163840
D = 256
H = 512
NCORE = 2
NSUB = 16
NW = NCORE * NSUB
RPW = NP // NW
SLACK = 256
EPP = EP + NW * SLACK
LCHUNK = 1024
NLCH = EP // LCHUNK

_mesh = plsc.VectorSubcoreMesh(core_axis_name="c", subcore_axis_name="s")
_lanes = jnp.float32


def _wid():
    return lax.axis_index("c") * NSUB + lax.axis_index("s")


def _ds(off, n):
    if isinstance(off, int):
        return pl.ds(off, n)
    return pl.ds(pl.multiple_of(off, 8), n)


def _vsum(v):
    s = v[0]
    for i in range(1, 16):
        s = s + v[i]
    return s


def _prefix_start(cnts_v, w):
    def body(u, acc):
        cu = _vsum(cnts_v[_ds(u * 16, 16)])
        return acc + jnp.where(u < w, cu, 0)
    p = lax.fori_loop(0, NW, body, jnp.int32(0))
    p8 = ((p + 7) // 8) * 8
    return w * SLACK + p8


def _mycnt(cnts_v, w):
    return _vsum(cnts_v[_ds(w * 16, 16)])


def _count_body(dst_hbm, out_hbm, dvec, cnt8, sem):
    w = _wid()
    r0 = w * RPW
    r1 = r0 + RPW

    def chunk(b, cv):
        pltpu.sync_copy(dst_hbm.at[_ds(b * LCHUNK, LCHUNK)], dvec)
        for k in range(LCHUNK // 16):
            dv = dvec[_ds(k * 16, 16)]
            m = jnp.logical_and(dv >= r0, dv < r1)
            cv = cv + jnp.where(m, 1, 0)
        return cv

    cv = lax.fori_loop(0, NLCH, chunk, jnp.zeros((16,), jnp.int32))
    cnt8[...] = cv
    pltpu.sync_copy(cnt8, out_hbm.at[_ds(w * 16, 16)])


_count_kernel = functools.partial(
    pl.kernel,
    out_type=jax.ShapeDtypeStruct((NW * 16,), jnp.int32),
    mesh=_mesh,
    scratch_types=[
        pltpu.VMEM((LCHUNK,), jnp.int32),
        pltpu.VMEM((16,), jnp.int32),
        pltpu.SemaphoreType.DMA,
    ],
)(_count_body)


def _part_body(src_hbm, dst_hbm, cnts_hbm, srcp_hbm, dstp_hbm, deg_hbm,
               cnts_v, svec, dvec, sels, seld, degv, sem):
    w = _wid()
    r0 = w * RPW
    r1 = r0 + RPW
    pltpu.sync_copy(cnts_hbm, cnts_v)
    start = _prefix_start(cnts_v, w)
    mycnt = _mycnt(cnts_v, w)
    iota16 = lax.iota(jnp.int32, 16)

    def zero(i, _):
        degv[_ds(i * 16, 16)] = jnp.zeros((16,), jnp.float32)
        return _
    lax.fori_loop(0, RPW // 16, zero, 0)

    def chunk(b, carry):
        cfill, pos = carry
        pltpu.sync_copy(src_hbm.at[_ds(b * LCHUNK, LCHUNK)], svec)
        pltpu.sync_copy(dst_hbm.at[_ds(b * LCHUNK, LCHUNK)], dvec)
        for k in range(LCHUNK // 16):
            dv = dvec[_ds(k * 16, 16)]
            sv = svec[_ds(k * 16, 16)]
            m = jnp.logical_and(dv >= r0, dv < r1)
            mi = jnp.where(m, 1, 0)
            run = jnp.int32(0)
            tgt = jnp.zeros((16,), jnp.int32)
            for l in range(16):
                tgt = tgt + jnp.where(iota16 == l, 1, 0) * (cfill + run)
                run = run + mi[l]
            nm = run
            plsc.store_scatter(seld, [tgt], dv - r0, mask=m)
            plsc.store_scatter(sels, [tgt], sv, mask=m)
            cfill = cfill + nm

            def flush(args):
                cf, p = args
                pltpu.sync_copy(sels.at[_ds(0, 128)],
                                srcp_hbm.at[_ds(start + p, 128)])
                pltpu.sync_copy(seld.at[_ds(0, 128)],
                                dstp_hbm.at[_ds(start + p, 128)])
                ts = sels[_ds(128, 16)]
                td = seld[_ds(128, 16)]
                sels[_ds(0, 16)] = ts
                seld[_ds(0, 16)] = td
                return (lax.convert_element_type(cf - 128, jnp.int32),
                        lax.convert_element_type(p + 128, jnp.int32))

            cfill, pos = lax.cond(
                cfill >= 128, flush,
                lambda a: (lax.convert_element_type(a[0], jnp.int32),
                           lax.convert_element_type(a[1], jnp.int32)),
                (cfill, pos))
        return (lax.convert_element_type(cfill, jnp.int32),
                lax.convert_element_type(pos, jnp.int32))

    cfill, pos = lax.fori_loop(0, NLCH, chunk,
                               (jnp.int32(0), jnp.int32(0)))
    pltpu.sync_copy(sels.at[_ds(0, 128)],
                    srcp_hbm.at[_ds(start + pos, 128)])
    pltpu.sync_copy(seld.at[_ds(0, 128)],
                    dstp_hbm.at[_ds(start + pos, 128)])

    nb = (mycnt + 127) // 128

    def degbatch(b, _):
        pltpu.sync_copy(dstp_hbm.at[_ds(start + b * 128, 128)],
                        dvec.at[_ds(0, 128)])

        def group(g, __):
            dv = dvec[_ds(g * 16, 16)]
            dv = jnp.clip(dv, 0, RPW - 1)
            valid = jnp.where((b * 128 + g * 16 + iota16) < mycnt, 1, 0)
            for l in range(16):
                r = dv[l]
                b16 = (r // 16) * 16
                lane = r - b16
                vv = degv[_ds(b16, 16)]
                vf = jnp.where(valid[l] == 1, 1.0, 0.0)
                degv[_ds(b16, 16)] = vv + jnp.where(
                    iota16 == lane, 1.0, 0.0) * vf
            return __

        lax.fori_loop(0, 8, group, 0)
        return _

    lax.fori_loop(0, nb, degbatch, 0)
    pltpu.sync_copy(degv, deg_hbm.at[_ds(r0, RPW)])


_part_kernel = functools.partial(
    pl.kernel,
    out_type=[
        jax.ShapeDtypeStruct((EPP,), jnp.int32),
        jax.ShapeDtypeStruct((EPP,), jnp.int32),
        jax.ShapeDtypeStruct((NP,), jnp.float32),
    ],
    mesh=_mesh,
    scratch_types=[
        pltpu.VMEM((NW * 16,), jnp.int32),
        pltpu.VMEM((LCHUNK,), jnp.int32),
        pltpu.VMEM((LCHUNK,), jnp.int32),
        pltpu.VMEM((160,), jnp.int32),
        pltpu.VMEM((160,), jnp.int32),
        pltpu.VMEM((RPW,), jnp.float32),
        pltpu.SemaphoreType.DMA,
    ],
)(_part_body)


def _make_segsum(ntab):
    def body(*refs):
        srcp_hbm, dstp_hbm, cnts_hbm = refs[0], refs[1], refs[2]
        tabs = refs[3:3 + ntab]
        outs = refs[3 + ntab:3 + 2 * ntab]
        cnts_v, idxs, idxd, rowbuf, acc, sem = refs[3 + 2 * ntab:]

        w = _wid()
        r0 = w * RPW
        pltpu.sync_copy(cnts_hbm, cnts_v)
        start = _prefix_start(cnts_v, w)
        mycnt = _mycnt(cnts_v, w)
        nb = (mycnt + 127) // 128
        iota16 = lax.iota(jnp.int32, 16)

        for t in range(ntab):
            def zero(i, _):
                acc[_ds(i * 16, 16)] = jnp.zeros((16,), jnp.float32)
                return _
            lax.fori_loop(0, RPW * 256 // 16, zero, 0)

            def batch(b, _):
                pltpu.sync_copy(srcp_hbm.at[_ds(start + b * 128, 128)],
                                idxs)
                pltpu.sync_copy(dstp_hbm.at[_ds(start + b * 128, 128)],
                                idxd)

                def group(g, __):
                    sv = jnp.clip(idxs[_ds(g * 16, 16)], 0, NP - 1)
                    dv = jnp.clip(idxd[_ds(g * 16, 16)], 0, RPW - 1)
                    pltpu.async_copy(tabs[t].at[sv], rowbuf, sem).wait()
                    valid = jnp.where((b * 128 + g * 16 + iota16) < mycnt,
                                      1, 0)
                    for l in range(16):
                        r = dv[l]
                        base = r * 256
                        for k in range(16):
                            a = acc[_ds(base + k * 16, 16)]
                            v = rowbuf[l, _ds(k * 16, 16)]
                            acc[_ds(base + k * 16, 16)] = a + jnp.where(
                                valid[l] == 1, v, 0.0)
                    return __

                lax.fori_loop(0, 8, group, 0)
                return _

            lax.fori_loop(0, nb, batch, 0)
            pltpu.sync_copy(
                acc,
                outs[t].at[_ds(r0 * 256, RPW * 256)])

    return functools.partial(
        pl.kernel,
        out_type=[jax.ShapeDtypeStruct((NP * 256,), jnp.float32)
                  for _ in range(ntab)],
        mesh=_mesh,
        scratch_types=[
            pltpu.VMEM((NW * 16,), jnp.int32),
            pltpu.VMEM((128,), jnp.int32),
            pltpu.VMEM((128,), jnp.int32),
            pltpu.VMEM((16, 256), jnp.float32),
            pltpu.VMEM((RPW * 256,), jnp.float32),
            pltpu.SemaphoreType.DMA,
        ],
    )(body)


_segsum1 = _make_segsum(1)
_segsum2 = _make_segsum(2)


def _head_body(src_hbm, dst_hbm, a_hbm, b_hbm, za_hbm, zb_hbm,
               idxs, idxd, buf1, buf2, sem_a, sem_b):
    w = _wid()
    base = w * (EP // NW)

    def chunk(j, _):
        pltpu.sync_copy(src_hbm.at[_ds(base + j * 64, 64)], idxs)
        pltpu.sync_copy(dst_hbm.at[_ds(base + j * 64, 64)], idxd)
        pltpu.async_copy(a_hbm.at[idxs], buf1, sem_a).wait()
        pltpu.async_copy(b_hbm.at[idxd], buf2, sem_b).wait()
        pltpu.sync_copy(buf1, za_hbm.at[_ds(base + j * 64, 64)])
        pltpu.sync_copy(buf2, zb_hbm.at[_ds(base + j * 64, 64)])
        return _

    lax.fori_loop(0, EP // NW // 64, chunk, 0)


_head_gather = functools.partial(
    pl.kernel,
    out_type=[
        jax.ShapeDtypeStruct((EP, H), jnp.float32),
        jax.ShapeDtypeStruct((EP, H), jnp.float32),
    ],
    mesh=_mesh,
    scratch_types=[
        pltpu.VMEM((64,), jnp.int32),
        pltpu.VMEM((64,), jnp.int32),
        pltpu.VMEM((64, H), jnp.float32),
        pltpu.VMEM((64, H), jnp.float32),
        pltpu.SemaphoreType.DMA,
        pltpu.SemaphoreType.DMA,
    ],
)(_head_body)


def _xs1_body(deg_ref, x_ref, out_ref, dinv_ref):
    dinv = lax.rsqrt(jnp.maximum(deg_ref[...] + 1.0, 1.0))
    out_ref[...] = x_ref[...] * dinv
    dinv_ref[...] = dinv


def _mm1_body(dinv_ref, s1_ref, x1_ref, w_ref, b_ref, oa_ref, ob_ref):
    dinv = dinv_ref[...]
    t = dinv * (s1_ref[...] + x1_ref[...])
    h = jnp.maximum(jnp.dot(t, w_ref[...],
                            preferred_element_type=jnp.float32) + b_ref[...],
                    0.0)
    x2 = h * dinv
    oa_ref[...] = x2[:, 0:256]
    ob_ref[...] = x2[:, 256:512]


def _mm2_body(dinv_ref, s2a_ref, s2b_ref, x2a_ref, x2b_ref, w_ref, b_ref,
              p1_ref, pb1_ref, a_ref, bb_ref):
    dinv = dinv_ref[...]
    t = jnp.concatenate(
        [dinv * (s2a_ref[...] + x2a_ref[...]),
         dinv * (s2b_ref[...] + x2b_ref[...])], axis=1)
    emb = jnp.dot(t, w_ref[...], preferred_element_type=jnp.float32) \
        + b_ref[...]
    p1 = p1_ref[...]
    a_ref[...] = jnp.dot(emb, p1[0:H],
                         preferred_element_type=jnp.float32) + pb1_ref[...]
    bb_ref[...] = jnp.dot(emb, p1[H:2 * H],
                          preferred_element_type=jnp.float32)


def _headmv_body(za_ref, zb_ref, p2_ref, pb2_ref, out_ref):
    z = jnp.maximum(za_ref[...] + zb_ref[...], 0.0)
    out_ref[...] = jnp.dot(z, p2_ref[...],
                           preferred_element_type=jnp.float32) + pb2_ref[...]


_RB = 256
_GRID_N = NP // _RB


def _tc_xs1(deg, x_pad):
    return pl.pallas_call(
        _xs1_body,
        grid=(_GRID_N,),
        in_specs=[
            pl.BlockSpec((_RB, 1), lambda i: (i, 0)),
            pl.BlockSpec((_RB, D), lambda i: (i, 0)),
        ],
        out_specs=[
            pl.BlockSpec((_RB, D), lambda i: (i, 0)),
            pl.BlockSpec((_RB, 1), lambda i: (i, 0)),
        ],
        out_shape=[
            jax.ShapeDtypeStruct((NP, D), jnp.float32),
            jax.ShapeDtypeStruct((NP, 1), jnp.float32),
        ],
    )(deg, x_pad)


def _tc_mm1(dinv, s1, x1, W1, b1):
    return pl.pallas_call(
        _mm1_body,
        grid=(_GRID_N,),
        in_specs=[
            pl.BlockSpec((_RB, 1), lambda i: (i, 0)),
            pl.BlockSpec((_RB, D), lambda i: (i, 0)),
            pl.BlockSpec((_RB, D), lambda i: (i, 0)),
            pl.BlockSpec((D, H), lambda i: (0, 0)),
            pl.BlockSpec((1, H), lambda i: (0, 0)),
        ],
        out_specs=[
            pl.BlockSpec((_RB, 256), lambda i: (i, 0)),
            pl.BlockSpec((_RB, 256), lambda i: (i, 0)),
        ],
        out_shape=[
            jax.ShapeDtypeStruct((NP, 256), jnp.float32),
            jax.ShapeDtypeStruct((NP, 256), jnp.float32),
        ],
    )(dinv, s1, x1, W1, b1)


def _tc_mm2(dinv, s2a, s2b, x2a, x2b, W2, b2, P1, pb1):
    return pl.pallas_call(
        _mm2_body,
        grid=(_GRID_N,),
        in_specs=[
            pl.BlockSpec((_RB, 1), lambda i: (i, 0)),
            pl.BlockSpec((_RB, 256), lambda i: (i, 0)),
            pl.BlockSpec((_RB, 256), lambda i: (i, 0)),
            pl.BlockSpec((_RB, 256), lambda i: (i, 0)),
            pl.BlockSpec((_RB, 256), lambda i: (i, 0)),
            pl.BlockSpec((H, H), lambda i: (0, 0)),
            pl.BlockSpec((1, H), lambda i: (0, 0)),
            pl.BlockSpec((2 * H, H), lambda i: (0, 0)),
            pl.BlockSpec((1, H), lambda i: (0, 0)),
        ],
        out_specs=[
            pl.BlockSpec((_RB, H), lambda i: (i, 0)),
            pl.BlockSpec((_RB, H), lambda i: (i, 0)),
        ],
        out_shape=[
            jax.ShapeDtypeStruct((NP, H), jnp.float32),
            jax.ShapeDtypeStruct((NP, H), jnp.float32),
        ],
    )(dinv, s2a, s2b, x2a, x2b, W2, b2, P1, pb1)


_EB = 1024


def _tc_head(Za, Zb, P2, pb2):
    return pl.pallas_call(
        _headmv_body,
        grid=(EP // _EB,),
        in_specs=[
            pl.BlockSpec((_EB, H), lambda i: (i, 0)),
            pl.BlockSpec((_EB, H), lambda i: (i, 0)),
            pl.BlockSpec((H, 1), lambda i: (0, 0)),
            pl.BlockSpec((1, 1), lambda i: (0, 0)),
        ],
        out_specs=pl.BlockSpec((_EB, 1), lambda i: (i, 0)),
        out_shape=jax.ShapeDtypeStruct((EP, 1), jnp.float32),
    )(Za, Zb, P2, pb2)


def kernel(x, edge_index, W1, b1, W2, b2, P1, pb1, P2, pb2):
    ei = edge_index.astype(jnp.int32)
    src = jnp.concatenate([ei[0], jnp.zeros((EP - E,), jnp.int32)])
    dst = jnp.concatenate([ei[1], jnp.full((EP - E,), N, jnp.int32)])

    x_pad = jnp.concatenate([x, jnp.zeros((NP - N, D), x.dtype)])

    deg = jnp.zeros((NP,), jnp.float32).at[dst].add(1.0)

    def segsum(t):
        return jnp.zeros_like(t).at[dst].add(t[src])

    xs1, dinv = _tc_xs1(deg.reshape(NP, 1), x_pad)
    s1 = segsum(xs1)
    x2a, x2b = _tc_mm1(dinv, s1, xs1, W1, b1.reshape(1, H))
    s2 = segsum(jnp.concatenate([x2a, x2b], axis=1))
    A, B = _tc_mm2(dinv, s2[:, :256], s2[:, 256:],
                   x2a, x2b, W2, b2.reshape(1, H), P1, pb1.reshape(1, H))
    Za, Zb = _head_gather(src, dst, A, B)
    pred = _tc_head(Za, Zb, P2, pb2.reshape(1, 1))
    return pred[:E, 0]
